# baseline (device time: 27530 ns/iter reference)
import jax
import jax.numpy as jnp
from jax import lax
from jax.experimental import pallas as pl
from jax.experimental.pallas import tpu as pltpu

N_DEV = 4


def kernel(x, w_mat):
    m_per, k = x.shape
    n = w_mat.shape[1]
    n_per = n // N_DEV

    def body(x_ref, w_ref, out_ref, send_buf, recv_buf, send_sems, recv_sems):
        my = lax.axis_index("i")

        barrier_sem = pltpu.get_barrier_semaphore()
        for h in range(1, N_DEV):
            pl.semaphore_signal(
                barrier_sem, inc=1,
                device_id=((my + h) % N_DEV,),
                device_id_type=pl.DeviceIdType.MESH,
            )

        x_f = x_ref[...]

        def block(peer):
            acc = jnp.dot(
                x_f,
                w_ref[:, pl.ds(peer * n_per, n_per)],
                preferred_element_type=jnp.float32,
            )
            return jnp.maximum(acc, 0.0)

        half = m_per // 2

        def start_chunk(h, peer, c):
            rdma = pltpu.make_async_remote_copy(
                src_ref=send_buf.at[h, pl.ds(c * half, half)],
                dst_ref=recv_buf.at[h, pl.ds(c * half, half)],
                send_sem=send_sems.at[h, c],
                recv_sem=recv_sems.at[h, c],
                device_id=(peer,),
                device_id_type=pl.DeviceIdType.MESH,
            )
            rdma.start()
            return rdma

        rdmas = []
        peer2 = (my + 2) % N_DEV
        w2 = w_ref[:, pl.ds(peer2 * n_per, n_per)]
        for c in (0, 1):
            acc = jnp.dot(
                x_f[c * half:(c + 1) * half, :], w2,
                preferred_element_type=jnp.float32,
            )
            send_buf[2, pl.ds(c * half, half), :] = (
                jnp.maximum(acc, 0.0).astype(jnp.bfloat16)
            )
            if c == 0:
                pl.semaphore_wait(barrier_sem, N_DEV - 1)
            rdmas.append(start_chunk(2, peer2, c))

        for h in (1, 3):
            peer = (my + h) % N_DEV
            send_buf[h, :, :] = block(peer).astype(jnp.bfloat16)
            for c in (0, 1):
                rdmas.append(start_chunk(h, peer, c))

        out_ref[pl.ds(my * m_per, m_per), :] = block(my)

        for h in (1, 3, 2):
            src = (my - h) % N_DEV
            for c in (0, 1):
                recv = pltpu.make_async_remote_copy(
                    src_ref=send_buf.at[h, pl.ds(c * half, half)],
                    dst_ref=recv_buf.at[h, pl.ds(c * half, half)],
                    send_sem=send_sems.at[h, c],
                    recv_sem=recv_sems.at[h, c],
                    device_id=(src,),
                    device_id_type=pl.DeviceIdType.MESH,
                )
                recv.wait_recv()
                out_ref[pl.ds(src * m_per + c * half, half), :] = (
                    recv_buf[h, pl.ds(c * half, half), :].astype(jnp.float32)
                )
        for rdma in rdmas:
            rdma.wait_send()

    out_shape = jax.ShapeDtypeStruct((N_DEV * m_per, n_per), jnp.float32)
    return pl.pallas_call(
        body,
        out_shape=out_shape,
        in_specs=[
            pl.BlockSpec(memory_space=pltpu.VMEM),
            pl.BlockSpec(memory_space=pltpu.VMEM),
        ],
        out_specs=pl.BlockSpec(memory_space=pltpu.VMEM),
        scratch_shapes=[
            pltpu.VMEM((N_DEV, m_per, n_per), jnp.bfloat16),
            pltpu.VMEM((N_DEV, m_per, n_per), jnp.bfloat16),
            pltpu.SemaphoreType.DMA((N_DEV, 2)),
            pltpu.SemaphoreType.DMA((N_DEV, 2)),
        ],
        compiler_params=pltpu.CompilerParams(collective_id=0),
    )(x, w_mat)


# device time: 27314 ns/iter; 1.0079x vs baseline; 1.0079x over previous
import jax
import jax.numpy as jnp
from jax import lax
from jax.experimental import pallas as pl
from jax.experimental.pallas import tpu as pltpu

N_DEV = 4


def kernel(x, w_mat):
    m_per, k = x.shape
    n = w_mat.shape[1]
    n_per = n // N_DEV

    def body(x_ref, w_ref, out_ref, send_buf, recv_buf, send_sems, recv_sems):
        my = lax.axis_index("i")

        barrier_sem = pltpu.get_barrier_semaphore()
        for h in range(1, N_DEV):
            pl.semaphore_signal(
                barrier_sem, inc=1,
                device_id=((my + h) % N_DEV,),
                device_id_type=pl.DeviceIdType.MESH,
            )

        x_f = x_ref[...]

        def block(peer):
            acc = jnp.dot(
                x_f,
                w_ref[:, pl.ds(peer * n_per, n_per)],
                preferred_element_type=jnp.float32,
            )
            return jnp.maximum(acc, 0.0)

        rdmas = []
        for h in (2, 1, 3):
            peer = (my + h) % N_DEV
            send_buf[h, :, :] = block(peer).astype(jnp.bfloat16)
            if h == 2:
                pl.semaphore_wait(barrier_sem, N_DEV - 1)
            rdma = pltpu.make_async_remote_copy(
                src_ref=send_buf.at[h],
                dst_ref=recv_buf.at[h],
                send_sem=send_sems.at[h],
                recv_sem=recv_sems.at[h],
                device_id=(peer,),
                device_id_type=pl.DeviceIdType.MESH,
            )
            rdma.start()
            rdmas.append(rdma)

        out_ref[pl.ds(my * m_per, m_per), :] = block(my)

        for h in (1, 3, 2):
            src = (my - h) % N_DEV
            recv = pltpu.make_async_remote_copy(
                src_ref=send_buf.at[h],
                dst_ref=recv_buf.at[h],
                send_sem=send_sems.at[h],
                recv_sem=recv_sems.at[h],
                device_id=(src,),
                device_id_type=pl.DeviceIdType.MESH,
            )
            recv.wait_recv()
            out_ref[pl.ds(src * m_per, m_per), :] = (
                recv_buf[h, :, :].astype(jnp.float32)
            )
        for rdma in rdmas:
            rdma.wait_send()

    out_shape = jax.ShapeDtypeStruct((N_DEV * m_per, n_per), jnp.float32)
    return pl.pallas_call(
        body,
        out_shape=out_shape,
        in_specs=[
            pl.BlockSpec(memory_space=pltpu.VMEM),
            pl.BlockSpec(memory_space=pltpu.VMEM),
        ],
        out_specs=pl.BlockSpec(memory_space=pltpu.VMEM),
        scratch_shapes=[
            pltpu.VMEM((N_DEV, m_per, n_per), jnp.bfloat16),
            pltpu.VMEM((N_DEV, m_per, n_per), jnp.bfloat16),
            pltpu.SemaphoreType.DMA((N_DEV,)),
            pltpu.SemaphoreType.DMA((N_DEV,)),
        ],
        compiler_params=pltpu.CompilerParams(collective_id=0),
    )(x, w_mat)
